# Initial kernel scaffold; baseline (speedup 1.0000x reference)
import jax
import jax.numpy as jnp
from jax.experimental import pallas as pl


def kernel(x, Wq, Wk, Wv, Wp, Wr, W1, W2):
    raise NotImplementedError("write your pallas kernel here")


# speedup vs baseline: 6.6892x; 6.6892x over previous
import functools

import jax
import jax.numpy as jnp
from jax.experimental import pallas as pl
from jax.experimental.pallas import tpu as pltpu

T, B, C, H, W = 4, 4, 192, 16, 16
E, TOPK, HID, HEADS = 8, 2, 768, 8
N = H * W
RT = B * N
R = T * RT
D = C // HEADS
F32 = jnp.float32


def _bn_rows(y):
    m = jnp.mean(y, axis=0, keepdims=True)
    c = y - m
    v = jnp.mean(c * c, axis=0, keepdims=True)
    return c * jax.lax.rsqrt(v + 1e-5)


def _lif4(y, tau):
    rows = y.shape[0] // T
    v = jnp.zeros((rows, y.shape[1]), F32)
    outs = []
    for t in range(T):
        xt = y[t * rows:(t + 1) * rows, :]
        v = v + (xt - v) / tau
        s = (v >= 1.0).astype(F32)
        v = v * (1.0 - s)
        outs.append(s)
    return jnp.concatenate(outs, axis=0)


def _ssa_router_kernel(x_ref, wqt_ref, wkt_ref, wvt_ref, wpt_ref, wrt_ref,
                       h_ref, wfull_ref, q_s, k_s, v_s, o_s):
    x = x_ref[...]
    s = _lif4(x, 2.0)
    dot = functools.partial(jnp.dot, preferred_element_type=F32)
    q_s[...] = _lif4(_bn_rows(dot(s, wqt_ref[...])), 2.0)
    k_s[...] = _lif4(_bn_rows(dot(s, wkt_ref[...])), 2.0)
    v_s[...] = _lif4(_bn_rows(dot(s, wvt_ref[...])), 2.0)

    def tb_body(i, carry):
        base = i * N
        q_tb = q_s[pl.ds(base, N), :]
        k_tb = k_s[pl.ds(base, N), :]
        v_tb = v_s[pl.ds(base, N), :]
        parts = []
        for hh in range(HEADS):
            sl = slice(hh * D, (hh + 1) * D)
            a = jax.lax.dot_general(
                q_tb[:, sl], k_tb[:, sl],
                (((1,), (1,)), ((), ())),
                preferred_element_type=F32) * 0.125
            parts.append(dot(a, v_tb[:, sl]))
        o_s[pl.ds(base, N), :] = jnp.concatenate(parts, axis=1)
        return carry

    jax.lax.fori_loop(0, T * B, tb_body, 0)

    o_sp = _lif4(o_s[...], 2.0)
    h = x + _bn_rows(dot(o_sp, wpt_ref[...]))
    h_ref[...] = h

    xm = 0.25 * (h[0:RT, :] + h[RT:2 * RT, :] + h[2 * RT:3 * RT, :]
                 + h[3 * RT:4 * RT, :])
    r = dot(xm, wrt_ref[...])
    rb = _bn_rows(r)
    logits = jnp.concatenate(
        [jnp.mean(rb[b * N:(b + 1) * N, :], axis=0, keepdims=True)
         for b in range(B)], axis=0)
    mx = jnp.max(logits, axis=-1, keepdims=True)
    ex = jnp.exp(logits - mx)
    probs = ex / jnp.sum(ex, axis=-1, keepdims=True)
    iota = jax.lax.broadcasted_iota(jnp.int32, (B, E), 1)
    m1 = jnp.max(probs, axis=-1, keepdims=True)
    i1 = jnp.min(jnp.where(probs == m1, iota, E), axis=-1, keepdims=True)
    oh1 = iota == i1
    pmasked = jnp.where(oh1, -1.0, probs)
    m2 = jnp.max(pmasked, axis=-1, keepdims=True)
    i2 = jnp.min(jnp.where(pmasked == m2, iota, E), axis=-1, keepdims=True)
    oh2 = iota == i2
    p1 = jnp.sum(jnp.where(oh1, probs, 0.0), axis=-1, keepdims=True)
    p2 = jnp.sum(jnp.where(oh2, probs, 0.0), axis=-1, keepdims=True)
    tot = p1 + p2
    wfull_ref[...] = jnp.where(oh1, p1 / tot, 0.0) + jnp.where(oh2, p2 / tot, 0.0)


def _experts_kernel(taus_ref, wfull_ref, h_ref, w1t_ref, w2t_ref, out_ref):
    e = pl.program_id(0)

    @pl.when(e == 0)
    def _init():
        out_ref[...] = h_ref[...]

    wb = [wfull_ref[b, e] for b in range(B)]
    selected = (wb[0] > 0) | (wb[1] > 0) | (wb[2] > 0) | (wb[3] > 0)

    @pl.when(selected)
    def _compute():
        tau = taus_ref[0, e]
        h = h_ref[...]
        dot = functools.partial(jnp.dot, preferred_element_type=F32)
        s = _lif4(h, tau)
        y1 = _bn_rows(dot(s, w1t_ref[0]))
        s2 = _lif4(y1, tau)
        yb = _bn_rows(dot(s2, w2t_ref[0]))
        for t in range(T):
            for b in range(B):
                lo = t * RT + b * N
                sl = slice(lo, lo + N)
                out_ref[sl, :] += wb[b] * yb[sl, :]


def kernel(x, Wq, Wk, Wv, Wp, Wr, W1, W2):
    x_r = x.reshape(T, B, C, N).transpose(0, 1, 3, 2).reshape(R, C)
    taus = jnp.linspace(1.5, 4.0, E, dtype=F32).reshape(1, E)

    h, wfull = pl.pallas_call(
        _ssa_router_kernel,
        out_shape=[jax.ShapeDtypeStruct((R, C), F32),
                   jax.ShapeDtypeStruct((B, E), F32)],
        scratch_shapes=[pltpu.VMEM((R, C), F32)] * 4,
    )(x_r, Wq.T, Wk.T, Wv.T, Wp.T, Wr.T)

    out = pl.pallas_call(
        _experts_kernel,
        grid=(E,),
        in_specs=[
            pl.BlockSpec(memory_space=pltpu.SMEM),
            pl.BlockSpec(memory_space=pltpu.SMEM),
            pl.BlockSpec((R, C), lambda e: (0, 0)),
            pl.BlockSpec((1, C, HID), lambda e: (e, 0, 0)),
            pl.BlockSpec((1, HID, C), lambda e: (e, 0, 0)),
        ],
        out_specs=pl.BlockSpec((R, C), lambda e: (0, 0)),
        out_shape=jax.ShapeDtypeStruct((R, C), F32),
    )(taus, wfull, h, W1.transpose(0, 2, 1), W2.transpose(0, 2, 1))

    return out.reshape(T, B, N, C).transpose(0, 1, 3, 2).reshape(T, B, C, H, W)
